# baseline (speedup 1.0000x reference)
import functools

import jax
import jax.numpy as jnp
from jax import lax
from jax.experimental import pallas as pl
from jax.experimental.pallas import tpu as pltpu
from jax.experimental.pallas import tpu_sc as plsc

_NC = 2
_NS = 16
_NW = _NC * _NS

_SCH = 32


@functools.cache
def _build(S, T, V, D):
    s_per_w = S // _NW
    chunks = s_per_w // _SCH
    assert chunks * _SCH == s_per_w and chunks >= 2

    mesh = plsc.VectorSubcoreMesh(
        core_axis_name="c", subcore_axis_name="s",
        num_cores=_NC, num_subcores=_NS)

    @functools.partial(
        pl.kernel,
        out_type=jax.ShapeDtypeStruct((T, D, S), jnp.float32),
        mesh=mesh,
        scratch_types=[
            pltpu.VMEM((2, T, _SCH), jnp.int32),
            pltpu.VMEM((T, _SCH, D), jnp.float32),
            pltpu.VMEM((T, D, _SCH), jnp.float32),
            pltpu.SemaphoreType.DMA,
            pltpu.SemaphoreType.DMA,
            pltpu.SemaphoreType.DMA,
            pltpu.SemaphoreType.DMA,
        ],
        compiler_params=pltpu.CompilerParams(
            use_tc_tiling_on_sc=False, needs_layout_passes=False),
    )
    def k(idx_hbm, table_hbm, out_hbm, idx_v, rows_v, st_v,
          isem0, isem1, osem, gsem):
        wid = lax.axis_index("s") * _NC + lax.axis_index("c")
        s0w = wid * s_per_w
        isems = (isem0, isem1)
        iota = lax.iota(jnp.int32, 16)
        iota2 = iota + 16

        def idx_copy(c, b):
            return pltpu.make_async_copy(
                idx_hbm.at[:, pl.ds(s0w + c * _SCH, _SCH)],
                idx_v.at[b], isems[b])

        def out_copy(c):
            return pltpu.make_async_copy(
                st_v, out_hbm.at[:, :, pl.ds(s0w + c * _SCH, _SCH)], osem)

        def gather_one(tcol, b):
            pltpu.async_copy(
                table_hbm.at[idx_v.at[b].at[tcol]], rows_v.at[tcol], gsem)

        def drain_one(_tcol, carry):
            pltpu.make_async_copy(
                table_hbm.at[idx_v.at[0].at[0]], rows_v.at[0], gsem).wait()
            return carry

        def transpose_one(tcol, carry):
            src = rows_v.at[tcol]
            dst = st_v.at[tcol]

            def tgroup(g, carry2):
                for u in range(4):
                    t = g * 4 + u
                    tfull = jnp.full((16,), t, jnp.int32)
                    v0 = src[t, pl.ds(0, 16)]
                    v1 = src[t, pl.ds(16, 16)]
                    plsc.store_scatter(dst, [iota, tfull], v0)
                    plsc.store_scatter(dst, [iota2, tfull], v1)
                return carry2

            lax.fori_loop(0, _SCH // 4, tgroup, 0)
            return carry

        idx_copy(0, 0).start()
        if chunks > 1:
            idx_copy(1, 1).start()

        for c in range(chunks):
            b = c % 2
            idx_copy(c, b).wait()
            lax.fori_loop(0, T, lambda t, cy: (gather_one(t, b), cy)[1], 0)
            if c > 0:
                out_copy(c - 1).wait()
            lax.fori_loop(0, T, drain_one, 0)
            lax.fori_loop(0, T, transpose_one, 0)
            out_copy(c).start()
            if c + 2 < chunks:
                idx_copy(c + 2, b).start()

        out_copy(chunks - 1).wait()

    return k


def kernel(token_ids, embed_matrix):
    S, T = token_ids.shape
    V, D = embed_matrix.shape
    idx_t = jnp.transpose(token_ids).astype(jnp.int32)
    out_t = _build(S, T, V, D)(idx_t, embed_matrix)
    return jnp.transpose(out_t, (2, 0, 1))


# speedup vs baseline: 1.3118x; 1.3118x over previous
import functools

import jax
import jax.numpy as jnp
from jax import lax
from jax.experimental import pallas as pl
from jax.experimental.pallas import tpu as pltpu
from jax.experimental.pallas import tpu_sc as plsc

_NC = 2
_NS = 16
_NW = _NC * _NS

_SCH = 16


@functools.cache
def _build(S, T, V, D):
    s_per_w = S // _NW
    chunks = s_per_w // _SCH
    assert chunks * _SCH == s_per_w and chunks >= 4

    mesh = plsc.VectorSubcoreMesh(
        core_axis_name="c", subcore_axis_name="s",
        num_cores=_NC, num_subcores=_NS)

    @functools.partial(
        pl.kernel,
        out_type=jax.ShapeDtypeStruct((T, D, S), jnp.float32),
        mesh=mesh,
        scratch_types=[
            pltpu.VMEM((2, T, _SCH), jnp.int32),
            pltpu.VMEM((2, T, _SCH, D), jnp.float32),
            pltpu.VMEM((2, T, D, _SCH), jnp.float32),
            pltpu.SemaphoreType.DMA,
            pltpu.SemaphoreType.DMA,
            pltpu.SemaphoreType.DMA,
            pltpu.SemaphoreType.DMA,
            pltpu.SemaphoreType.DMA,
            pltpu.SemaphoreType.DMA,
        ],
        compiler_params=pltpu.CompilerParams(
            use_tc_tiling_on_sc=False, needs_layout_passes=False),
    )
    def k(idx_hbm, table_hbm, out_hbm, idx_v, rows_v, st_v,
          isem0, isem1, osem0, osem1, gsem0, gsem1):
        wid = lax.axis_index("s") * _NC + lax.axis_index("c")
        s0w = wid * s_per_w
        isems = (isem0, isem1)
        osems = (osem0, osem1)
        gsems = (gsem0, gsem1)
        iota = lax.iota(jnp.int32, 16)
        iota2 = iota + 16

        def idx_copy(c, b):
            return pltpu.make_async_copy(
                idx_hbm.at[:, pl.ds(s0w + c * _SCH, _SCH)],
                idx_v.at[b], isems[b])

        def out_copy(c, p):
            return pltpu.make_async_copy(
                st_v.at[p],
                out_hbm.at[:, :, pl.ds(s0w + c * _SCH, _SCH)], osems[p])

        def fire_gathers(b):
            def one(tcol, cy):
                pltpu.async_copy(
                    table_hbm.at[idx_v.at[b].at[tcol]],
                    rows_v.at[b].at[tcol], gsems[b])
                return cy
            lax.fori_loop(0, T, one, 0)

        def drain_gathers(p):
            def one(_t, cy):
                pltpu.make_async_copy(
                    table_hbm.at[idx_v.at[0].at[0]],
                    rows_v.at[0].at[0], gsems[p]).wait()
                return cy
            lax.fori_loop(0, T, one, 0)

        def transpose_chunk(p):
            def per_tcol(tcol, cy):
                src = rows_v.at[p].at[tcol]
                dst = st_v.at[p].at[tcol]

                def grp(g, cy2):
                    for u in range(4):
                        t = g * 4 + u
                        ts = jnp.full((16,), t, jnp.int32)
                        plsc.store_scatter(dst, [iota, ts],
                                           src[t, pl.ds(0, 16)])
                        plsc.store_scatter(dst, [iota2, ts],
                                           src[t, pl.ds(16, 16)])
                    return cy2

                lax.fori_loop(0, _SCH // 4, grp, 0)
                return cy
            lax.fori_loop(0, T, per_tcol, 0)

        idx_copy(0, 0).start()
        idx_copy(1, 1).start()

        for c in range(chunks + 1):
            b = c % 2
            if c < chunks:
                idx_copy(c, b).wait()
                fire_gathers(b)
            if c >= 1:
                p = (c - 1) % 2
                if c >= 3:
                    out_copy(c - 3, p).wait()
                drain_gathers(p)
                if c + 1 < chunks:
                    idx_copy(c + 1, p).start()
                transpose_chunk(p)
                out_copy(c - 1, p).start()

        out_copy(chunks - 2, chunks % 2).wait()
        out_copy(chunks - 1, (chunks - 1) % 2).wait()

    return k


def kernel(token_ids, embed_matrix):
    S, T = token_ids.shape
    V, D = embed_matrix.shape
    idx_t = jnp.transpose(token_ids).astype(jnp.int32)
    out_t = _build(S, T, V, D)(idx_t, embed_matrix)
    return jnp.transpose(out_t, (2, 0, 1))


# speedup vs baseline: 1.3138x; 1.0016x over previous
import functools

import jax
import jax.numpy as jnp
from jax import lax
from jax.experimental import pallas as pl
from jax.experimental.pallas import tpu as pltpu
from jax.experimental.pallas import tpu_sc as plsc

_NC = 2
_NS = 16
_NW = _NC * _NS

_SCH = 16


@functools.cache
def _build(S, T, V, D):
    s_per_w = S // _NW
    chunks = s_per_w // _SCH
    assert chunks * _SCH == s_per_w and chunks >= 4

    mesh = plsc.VectorSubcoreMesh(
        core_axis_name="c", subcore_axis_name="s",
        num_cores=_NC, num_subcores=_NS)

    @functools.partial(
        pl.kernel,
        out_type=jax.ShapeDtypeStruct((T, D, S), jnp.float32),
        mesh=mesh,
        scratch_types=[
            pltpu.VMEM((2, T, _SCH), jnp.int32),
            pltpu.VMEM((2, T, _SCH, D), jnp.float32),
            pltpu.VMEM((2, T, D, _SCH), jnp.float32),
            pltpu.SemaphoreType.DMA,
            pltpu.SemaphoreType.DMA,
            pltpu.SemaphoreType.DMA,
            pltpu.SemaphoreType.DMA,
            pltpu.SemaphoreType.DMA,
            pltpu.SemaphoreType.DMA,
        ],
        compiler_params=pltpu.CompilerParams(
            use_tc_tiling_on_sc=False, needs_layout_passes=False),
    )
    def k(idx_hbm, table_hbm, out_hbm, idx_v, rows_v, st_v,
          isem0, isem1, osem0, osem1, gsem0, gsem1):
        wid = lax.axis_index("s") * _NC + lax.axis_index("c")
        s0w = wid * s_per_w
        isems = (isem0, isem1)
        osems = (osem0, osem1)
        gsems = (gsem0, gsem1)
        iota = lax.iota(jnp.int32, 16)
        iota2 = iota + 16

        def idx_copy(c, b):
            return pltpu.make_async_copy(
                idx_hbm.at[:, pl.ds(s0w + c * _SCH, _SCH)],
                idx_v.at[b], isems[b])

        def out_copy(c, p):
            return pltpu.make_async_copy(
                st_v.at[p],
                out_hbm.at[:, :, pl.ds(s0w + c * _SCH, _SCH)], osems[p])

        def fire_gathers(b):
            def one(tcol, cy):
                pltpu.async_copy(
                    table_hbm.at[idx_v.at[b].at[tcol]],
                    rows_v.at[b].at[tcol], gsems[b])
                return cy
            lax.fori_loop(0, T, one, 0)

        def drain_gathers(p):
            def one(_t, cy):
                pltpu.make_async_copy(
                    table_hbm.at[idx_v.at[0].at[0]],
                    rows_v.at[0].at[0], gsems[p]).wait()
                return cy
            lax.fori_loop(0, T, one, 0)

        def transpose_chunk(p):
            def per_tcol(tcol, cy):
                src = rows_v.at[p].at[tcol]
                dst = st_v.at[p].at[tcol]

                def grp(g, cy2):
                    for u in range(4):
                        t = g * 4 + u
                        ts = jnp.full((16,), t, jnp.int32)
                        plsc.store_scatter(dst, [iota, ts],
                                           src[t, pl.ds(0, 16)])
                        plsc.store_scatter(dst, [iota2, ts],
                                           src[t, pl.ds(16, 16)])
                    return cy2

                lax.fori_loop(0, _SCH // 4, grp, 0)
                return cy
            lax.fori_loop(0, T, per_tcol, 0)

        idx_copy(0, 0).start()
        idx_copy(1, 1).start()

        for c in range(chunks + 1):
            b = c % 2
            if c < chunks:
                idx_copy(c, b).wait()
                fire_gathers(b)
            if c >= 1:
                p = (c - 1) % 2
                if c >= 3:
                    out_copy(c - 3, p).wait()
                drain_gathers(p)
                if c + 1 < chunks:
                    idx_copy(c + 1, p).start()
                transpose_chunk(p)
                out_copy(c - 1, p).start()

        out_copy(chunks - 2, chunks % 2).wait()
        out_copy(chunks - 1, (chunks - 1) % 2).wait()

    return k


def kernel(token_ids, embed_matrix):
    S, T = token_ids.shape
    V, D = embed_matrix.shape
    idx_t = jnp.transpose(token_ids).astype(jnp.int32)
    t128 = lax.optimization_barrier(embed_matrix.reshape(V * D // 128, 128))
    table_lin = t128.reshape(V, D)
    out_t = _build(S, T, V, D)(idx_t, table_lin)
    return jnp.transpose(out_t, (2, 0, 1))
